# baseline (speedup 1.0000x reference)
import functools

import jax
import jax.numpy as jnp
from jax import lax
from jax.experimental import pallas as pl
from jax.experimental.pallas import tpu as pltpu
from jax.experimental.pallas import tpu_sc as plsc

_EMBED_DIM = 32
_NBUF = 8
_LOOKAHEAD = 6


def _make_gather(B, D, num_workers, chunk):
    assert B % (num_workers * chunk) == 0
    b_per_w = B // num_workers
    n = b_per_w // chunk
    assert n % _NBUF == 0 and n >= 3 * _NBUF
    mesh = plsc.VectorSubcoreMesh(core_axis_name="c", subcore_axis_name="s")

    @functools.partial(
        pl.kernel,
        mesh=mesh,
        out_type=jax.ShapeDtypeStruct((B, D), jnp.float32),
        compiler_params=pltpu.CompilerParams(use_tc_tiling_on_sc=False),
        scratch_types=(
            [pltpu.VMEM((b_per_w,), jnp.int32)]
            + [pltpu.VMEM((chunk, D), jnp.float32) for _ in range(_NBUF)]
            + [pltpu.SemaphoreType.DMA for _ in range(2 * _NBUF)]
        ),
    )
    def gather_kernel(idx_hbm, table_hbm, out_hbm, idx_all, *bufs_and_sems):
        rows = bufs_and_sems[:_NBUF]
        gsem = bufs_and_sems[_NBUF : 2 * _NBUF]
        ssem = bufs_and_sems[2 * _NBUF :]

        num_cores = lax.axis_size("c")
        wid = lax.axis_index("s") * num_cores + lax.axis_index("c")
        base = wid * b_per_w

        pltpu.sync_copy(idx_hbm.at[pl.ds(base, b_per_w)], idx_all)

        def gather_desc(i, b):
            return pltpu.make_async_copy(
                table_hbm.at[idx_all.at[pl.ds(i * chunk, chunk)]], rows[b], gsem[b]
            )

        def store_desc(i, b):
            return pltpu.make_async_copy(
                rows[b], out_hbm.at[pl.ds(base + i * chunk, chunk)], ssem[b]
            )

        def step(i, b):
            j = i + _LOOKAHEAD
            if isinstance(j, int) and j >= n:
                pass
            else:
                bj = (b + _LOOKAHEAD) % _NBUF
                if not (isinstance(j, int) and j < _NBUF):
                    store_desc(j - _NBUF, bj).wait()
                gather_desc(j, bj).start()
            gather_desc(i, b).wait()
            store_desc(i, b).start()

        for j in range(_LOOKAHEAD):
            gather_desc(j, j % _NBUF).start()
        for i in range(_NBUF):
            step(i, i % _NBUF)

        def group(g, carry):
            for b in range(_NBUF):
                step(g * _NBUF + b, b)
            return carry

        lax.fori_loop(1, n // _NBUF - 1, group, 0)

        for i in range(n - _NBUF, n):
            step(i, i % _NBUF)
        for b in range(_NBUF):
            i = n - _NBUF + b
            store_desc(i, b).wait()

    return gather_kernel


def _retile_g128(G128, FG, Bdim, bk):
    nb = Bdim // bk

    def body(g_ref, o_ref):
        o_ref[...] = g_ref[...]

    return pl.pallas_call(
        body,
        grid=(FG, nb),
        in_specs=[pl.BlockSpec((bk, 128), lambda fg, c: (fg * nb + c, 0))],
        out_specs=pl.BlockSpec((bk, 128), lambda fg, c: (c, fg)),
        out_shape=jax.ShapeDtypeStruct((Bdim, FG * 128), jnp.float32),
    )(G128)


@jax.jit
def kernel(x, W):
    Bdim, F = x.shape
    D = _EMBED_DIM
    FG = F // 4
    idx = x.reshape(Bdim, FG, 4).transpose(1, 0, 2).reshape(-1).astype(jnp.int32)
    G = _make_gather(Bdim * F, D, 32, 256)(idx, W)
    return G.reshape(FG, Bdim, 4, D).transpose(1, 0, 2, 3).reshape(Bdim, F, D)


# speedup vs baseline: 2.0039x; 2.0039x over previous
import functools

import jax
import jax.numpy as jnp
from jax import lax
from jax.experimental import pallas as pl
from jax.experimental.pallas import tpu as pltpu
from jax.experimental.pallas import tpu_sc as plsc

_EMBED_DIM = 32
_NBUF = 8
_LOOKAHEAD = 6
_CHUNK = 256


def _make_gather(B, F, D, num_workers):
    N = B * F
    per_w = N // num_workers
    n = per_w // _CHUNK
    assert B % _CHUNK == 0 and per_w % _CHUNK == 0
    assert n % _NBUF == 0 and n >= 3 * _NBUF
    mesh = plsc.VectorSubcoreMesh(core_axis_name="c", subcore_axis_name="s")

    @functools.partial(
        pl.kernel,
        mesh=mesh,
        out_type=jax.ShapeDtypeStruct((B, F * D), jnp.float32),
        compiler_params=pltpu.CompilerParams(use_tc_tiling_on_sc=False),
        scratch_types=(
            [pltpu.VMEM((per_w,), jnp.int32)]
            + [pltpu.VMEM((_CHUNK, D), jnp.float32) for _ in range(_NBUF)]
            + [pltpu.SemaphoreType.DMA for _ in range(2 * _NBUF)]
        ),
    )
    def gather_kernel(idx_hbm, table_hbm, out_hbm, idx_all, *bufs_and_sems):
        rows = bufs_and_sems[:_NBUF]
        gsem = bufs_and_sems[_NBUF : 2 * _NBUF]
        ssem = bufs_and_sems[2 * _NBUF :]

        num_cores = lax.axis_size("c")
        wid = lax.axis_index("s") * num_cores + lax.axis_index("c")
        base = wid * per_w

        pltpu.sync_copy(idx_hbm.at[pl.ds(base, per_w)], idx_all)

        def gather_desc(i, b):
            return pltpu.make_async_copy(
                table_hbm.at[idx_all.at[pl.ds(i * _CHUNK, _CHUNK)]],
                rows[b],
                gsem[b],
            )

        def store_desc(i, b):
            j0 = base + i * _CHUNK
            f = j0 // B
            b0 = j0 % B
            return pltpu.make_async_copy(
                rows[b],
                out_hbm.at[pl.ds(b0, _CHUNK), pl.ds(f * D, D)],
                ssem[b],
            )

        def step(i, b):
            j = i + _LOOKAHEAD
            if isinstance(j, int) and j >= n:
                pass
            else:
                bj = (b + _LOOKAHEAD) % _NBUF
                if not (isinstance(j, int) and j < _NBUF):
                    store_desc(j - _NBUF, bj).wait()
                gather_desc(j, bj).start()
            gather_desc(i, b).wait()
            store_desc(i, b).start()

        for j in range(_LOOKAHEAD):
            gather_desc(j, j % _NBUF).start()
        for i in range(_NBUF):
            step(i, i % _NBUF)

        def group(g, carry):
            for b in range(_NBUF):
                step(g * _NBUF + b, b)
            return carry

        lax.fori_loop(1, n // _NBUF - 1, group, 0)

        for i in range(n - _NBUF, n):
            step(i, i % _NBUF)
        for b in range(_NBUF):
            i = n - _NBUF + b
            store_desc(i, b).wait()

    return gather_kernel


@jax.jit
def kernel(x, W):
    Bdim, F = x.shape
    D = _EMBED_DIM
    idx = x.T.reshape(-1).astype(jnp.int32)
    out2d = _make_gather(Bdim, F, D, 32)(idx, W)
    return out2d.reshape(Bdim, F, D)
